# Initial kernel scaffold; baseline (speedup 1.0000x reference)
import jax
import jax.numpy as jnp
from jax.experimental import pallas as pl


def kernel(x, Wq, bq, Wk, bk, Wv, bv, Wo, bo, qn_w, kn_w, edge_bias):
    raise NotImplementedError("write your pallas kernel here")


# speedup vs baseline: 12.3830x; 12.3830x over previous
import functools
import math

import jax
import jax.numpy as jnp
from jax.experimental import pallas as pl

B = 1
L = 4096
QUERY_DIM = 1024
HEADS = 16
DIM_HEAD = 64
NUM_FRAMES = 8
NUM_SPATIAL = 12
NUM_TEMPORAL = 4
MAX_SPATIAL_LEN = 2048
EPS = 1e-6

T = NUM_FRAMES
S = L // T
INNER = HEADS * DIM_HEAD

HG = 4
G = HEADS // HG
HGD = HG * DIM_HEAD

ROW_BLK = 512


def _seq_shifts(n, max_len):
    shifts = [0]
    s = 1
    while len(shifts) < n and s < max_len:
        shifts.append(s)
        if len(shifts) < n:
            shifts.append(-s)
        s *= 2
    return shifts[:n]


def _temporal_shifts(n):
    shifts = []
    s = 1
    while len(shifts) < n:
        shifts.append(s)
        if len(shifts) < n:
            shifts.append(-s)
        s *= 2
    return shifts[:n]


SPATIAL_SHIFTS = _seq_shifts(NUM_SPATIAL, MAX_SPATIAL_LEN)
TEMPORAL_SHIFTS = _temporal_shifts(NUM_TEMPORAL)
TOTAL_EDGES = NUM_SPATIAL + NUM_TEMPORAL


def _proj_kernel(x_ref, wq_ref, wk_ref, wv_ref, bq_ref, bk_ref, bv_ref,
                 qnw_ref, knw_ref, q_ref, k_ref, v_ref):
    xb = x_ref[...]
    q = jnp.dot(xb, wq_ref[...], preferred_element_type=jnp.float32) + bq_ref[...]
    k = jnp.dot(xb, wk_ref[...], preferred_element_type=jnp.float32) + bk_ref[...]
    v = jnp.dot(xb, wv_ref[...], preferred_element_type=jnp.float32) + bv_ref[...]
    qm = jnp.mean(q * q, axis=-1, keepdims=True)
    km = jnp.mean(k * k, axis=-1, keepdims=True)
    q_ref[...] = q * jax.lax.rsqrt(qm + EPS) * qnw_ref[...]
    k_ref[...] = k * jax.lax.rsqrt(km + EPS) * knw_ref[...]
    v_ref[...] = v


def _shift_rows(x, s):
    n = x.shape[0]
    s = s % n
    if s == 0:
        return x
    hi = jax.lax.slice_in_dim(x, s, n, axis=0)
    lo = jax.lax.slice_in_dim(x, 0, s, axis=0)
    return jax.lax.concatenate([hi, lo], dimension=0)


def _attn_kernel(q_ref, k_ref, v_ref, eb_ref, out_ref):
    g = pl.program_id(0)
    r = pl.program_id(1)
    scale = DIM_HEAD ** (-0.5)

    qf = q_ref[...] * scale
    base = r * S
    kf = k_ref[pl.ds(base, S), :]
    vf = v_ref[pl.ds(base, S), :]

    d_idx = jax.lax.broadcasted_iota(jnp.int32, (HGD, HG), 0) // DIM_HEAD
    h_idx = jax.lax.broadcasted_iota(jnp.int32, (HGD, HG), 1)
    mask = (d_idx == h_idx).astype(jnp.float32)

    ebg = eb_ref[pl.ds(g * HG, HG), :]

    scores = []
    for s in SPATIAL_SHIFTS:
        kr = _shift_rows(kf, s)
        sc = jnp.dot(qf * kr, mask, preferred_element_type=jnp.float32)
        scores.append(sc)
    for dt in TEMPORAL_SHIFTS:
        t2 = jax.lax.rem(r + dt + T, T)
        kr = k_ref[pl.ds(t2 * S, S), :]
        sc = jnp.dot(qf * kr, mask, preferred_element_type=jnp.float32)
        scores.append(sc)
    for e in range(TOTAL_EDGES):
        scores[e] = scores[e] + ebg[:, e].reshape(1, HG)

    m = scores[0]
    for e in range(1, TOTAL_EDGES):
        m = jnp.maximum(m, scores[e])
    probs = [jnp.exp(sc - m) for sc in scores]
    z = probs[0]
    for e in range(1, TOTAL_EDGES):
        z = z + probs[e]
    inv_z = 1.0 / z

    acc = jnp.zeros((S, HGD), jnp.float32)
    for e, s in enumerate(SPATIAL_SHIFTS):
        vr = _shift_rows(vf, s)
        w = jnp.dot(probs[e] * inv_z, mask.T, preferred_element_type=jnp.float32)
        acc = acc + w * vr
    for i, dt in enumerate(TEMPORAL_SHIFTS):
        e = NUM_SPATIAL + i
        t2 = jax.lax.rem(r + dt + T, T)
        vr = v_ref[pl.ds(t2 * S, S), :]
        w = jnp.dot(probs[e] * inv_z, mask.T, preferred_element_type=jnp.float32)
        acc = acc + w * vr

    out_ref[...] = acc


def _out_kernel(c_ref, wo_ref, bo_ref, o_ref):
    o_ref[...] = jnp.dot(c_ref[...], wo_ref[...],
                         preferred_element_type=jnp.float32) + bo_ref[...]


@functools.partial(jax.jit, static_argnames=())
def kernel(x, Wq, bq, Wk, bk, Wv, bv, Wo, bo, qn_w, kn_w, edge_bias):
    x2 = x.reshape(L, QUERY_DIM)
    bq2 = bq.reshape(1, INNER)
    bk2 = bk.reshape(1, INNER)
    bv2 = bv.reshape(1, INNER)
    bo2 = bo.reshape(1, QUERY_DIM)
    qnw2 = qn_w.reshape(1, INNER)
    knw2 = kn_w.reshape(1, INNER)

    n_row_blocks = L // ROW_BLK
    q, k, v = pl.pallas_call(
        _proj_kernel,
        grid=(n_row_blocks,),
        in_specs=[
            pl.BlockSpec((ROW_BLK, QUERY_DIM), lambda i: (i, 0)),
            pl.BlockSpec((QUERY_DIM, INNER), lambda i: (0, 0)),
            pl.BlockSpec((QUERY_DIM, INNER), lambda i: (0, 0)),
            pl.BlockSpec((QUERY_DIM, INNER), lambda i: (0, 0)),
            pl.BlockSpec((1, INNER), lambda i: (0, 0)),
            pl.BlockSpec((1, INNER), lambda i: (0, 0)),
            pl.BlockSpec((1, INNER), lambda i: (0, 0)),
            pl.BlockSpec((1, INNER), lambda i: (0, 0)),
            pl.BlockSpec((1, INNER), lambda i: (0, 0)),
        ],
        out_specs=[
            pl.BlockSpec((ROW_BLK, INNER), lambda i: (i, 0)),
            pl.BlockSpec((ROW_BLK, INNER), lambda i: (i, 0)),
            pl.BlockSpec((ROW_BLK, INNER), lambda i: (i, 0)),
        ],
        out_shape=[jax.ShapeDtypeStruct((L, INNER), jnp.float32)] * 3,
    )(x2, Wq, Wk, Wv, bq2, bk2, bv2, qnw2, knw2)

    combined = pl.pallas_call(
        _attn_kernel,
        grid=(G, T),
        in_specs=[
            pl.BlockSpec((S, HGD), lambda g, r: (r, g)),
            pl.BlockSpec((L, HGD), lambda g, r: (0, g)),
            pl.BlockSpec((L, HGD), lambda g, r: (0, g)),
            pl.BlockSpec((HEADS, TOTAL_EDGES), lambda g, r: (0, 0)),
        ],
        out_specs=pl.BlockSpec((S, HGD), lambda g, r: (r, g)),
        out_shape=jax.ShapeDtypeStruct((L, INNER), jnp.float32),
    )(q, k, v, edge_bias)

    out = pl.pallas_call(
        _out_kernel,
        grid=(n_row_blocks,),
        in_specs=[
            pl.BlockSpec((ROW_BLK, INNER), lambda i: (i, 0)),
            pl.BlockSpec((INNER, QUERY_DIM), lambda i: (0, 0)),
            pl.BlockSpec((1, QUERY_DIM), lambda i: (0, 0)),
        ],
        out_specs=pl.BlockSpec((ROW_BLK, QUERY_DIM), lambda i: (i, 0)),
        out_shape=jax.ShapeDtypeStruct((L, QUERY_DIM), jnp.float32),
    )(combined, Wo, bo2)

    return out.reshape(B, L, QUERY_DIM)
